# Initial kernel scaffold; baseline (speedup 1.0000x reference)
import jax
import jax.numpy as jnp
from jax.experimental import pallas as pl


def kernel(logits, alive_log_probs, alive_seq):
    raise NotImplementedError("write your pallas kernel here")


# speedup vs baseline: 2.2114x; 2.2114x over previous
import functools

import jax
import jax.numpy as jnp
from jax.experimental import pallas as pl
from jax.experimental.pallas import tpu as pltpu

_EOS_ID = 2
_NEG_INF = 1.0e7


def _step_kernel(logits_ref, alp_ref, seq_ref, seq_out_ref, lp_out_ref, *, beams_to_keep):
    x = logits_ref[0]
    beam, V = x.shape
    alp = alp_ref[0, 0]
    cur_len = seq_ref.shape[2]

    rowmax = jnp.max(x, axis=1, keepdims=True)
    lse = rowmax + jnp.log(jnp.sum(jnp.exp(x - rowmax), axis=1, keepdims=True))
    adj = x - lse + alp[:, None]

    flat_iota = (jax.lax.broadcasted_iota(jnp.int32, (beam, V), 0) * V
                 + jax.lax.broadcasted_iota(jnp.int32, (beam, V), 1))

    INT_BIG = jnp.int32(2**31 - 1)
    work = adj
    cand_val = []
    cand_idx = []
    for _ in range(beams_to_keep):
        m = jnp.max(work)
        fi = jnp.min(jnp.where(work == m, flat_iota, INT_BIG))
        cand_val.append(m)
        cand_idx.append(fi)
        work = jnp.where(flat_iota == fi, -jnp.inf, work)

    cand_id = [fi % V for fi in cand_idx]
    cand_beam = [fi // V for fi in cand_idx]
    masked = [v + jnp.where(i == _EOS_ID, jnp.float32(-_NEG_INF), jnp.float32(0.0))
              for v, i in zip(cand_val, cand_id)]

    taken = [jnp.bool_(False)] * beams_to_keep
    sel_val = []
    sel_beam = []
    sel_id = []
    for _ in range(beam):
        best_v = jnp.float32(-jnp.inf)
        best_b = jnp.int32(0)
        best_i = jnp.int32(0)
        best_j = jnp.int32(-1)
        for j in range(beams_to_keep):
            cond = jnp.logical_and(jnp.logical_not(taken[j]), masked[j] > best_v)
            best_v = jnp.where(cond, masked[j], best_v)
            best_b = jnp.where(cond, cand_beam[j], best_b)
            best_i = jnp.where(cond, cand_id[j], best_i)
            best_j = jnp.where(cond, jnp.int32(j), best_j)
        taken = [jnp.logical_or(taken[j], best_j == j) for j in range(beams_to_keep)]
        sel_val.append(best_v)
        sel_beam.append(best_b)
        sel_id.append(best_i)

    row_iota = jax.lax.broadcasted_iota(jnp.int32, (beam, 1), 0)
    lp_vec = jnp.zeros((1, beam), jnp.float32)
    col_iota = jax.lax.broadcasted_iota(jnp.int32, (1, beam), 1)
    for k in range(beam):
        lp_vec = jnp.where(col_iota == k, sel_val[k], lp_vec)
    lp_out_ref[0] = lp_vec

    seq = seq_ref[0]
    out_seq = jnp.zeros((beam, cur_len + 1), jnp.int32)
    pos_iota = jax.lax.broadcasted_iota(jnp.int32, (beam, cur_len + 1), 1)
    for k in range(beam):
        src = jnp.zeros((1, cur_len), jnp.int32)
        for b in range(beam):
            src = jnp.where(sel_beam[k] == b, seq[b:b + 1, :], src)
        row_full = jnp.concatenate([src, jnp.full((1, 1), 0, jnp.int32)], axis=1)
        row_full = jnp.where(pos_iota[:1] == cur_len, sel_id[k], row_full)
        out_seq = jnp.where(row_iota == k, row_full, out_seq)
    seq_out_ref[0] = out_seq


def kernel(logits, alive_log_probs, alive_seq):
    batch, beam, V = logits.shape
    cur_len = alive_seq.shape[2]
    alp3 = alive_log_probs.reshape(batch, 1, beam)

    grid = (batch,)
    seq_out, lp_out = pl.pallas_call(
        functools.partial(_step_kernel, beams_to_keep=2 * beam),
        grid=grid,
        in_specs=[
            pl.BlockSpec((1, beam, V), lambda b: (b, 0, 0)),
            pl.BlockSpec((1, 1, beam), lambda b: (b, 0, 0)),
            pl.BlockSpec((1, beam, cur_len), lambda b: (b, 0, 0)),
        ],
        out_specs=[
            pl.BlockSpec((1, beam, cur_len + 1), lambda b: (b, 0, 0)),
            pl.BlockSpec((1, 1, beam), lambda b: (b, 0, 0)),
        ],
        out_shape=[
            jax.ShapeDtypeStruct((batch, beam, cur_len + 1), jnp.int32),
            jax.ShapeDtypeStruct((batch, 1, beam), jnp.float32),
        ],
    )(logits, alp3, alive_seq)
    return seq_out, lp_out.reshape(batch, beam)


# speedup vs baseline: 9.0460x; 4.0906x over previous
import functools

import jax
import jax.numpy as jnp
from jax import lax
from jax.experimental import pallas as pl
from jax.experimental.pallas import tpu as pltpu
from jax.experimental.pallas import tpu_sc as plsc

_EOS_ID = 2
_NEG_INF = 1.0e7

_BATCH = 32
_BEAM = 4
_VOCAB = 100000
_K2 = 8

_CH = 2000
_NCH = _VOCAB // _CH
_U = 5
_NEG = -3.0e38
_BIGI = 2**31 - 1


def _splat_f(x):
    return jnp.full((16,), x, jnp.float32)


def _splat_i(x):
    return jnp.full((16,), x, jnp.int32)


def _worker_id():
    return lax.axis_index("s") * 2 + lax.axis_index("c")


def _sc_body(logits, out_v, out_f, out_gs, rowbuf, cmax, stage_v, stage_f, stage_gs):
    b = _worker_id()
    iota16 = lax.iota(jnp.int32, 16)

    vals_all = []
    flats_all = []
    g_list = []
    s_list = []

    for beam in range(_BEAM):
        pltpu.sync_copy(logits.at[b, beam], rowbuf)

        def chunk_body(c, G):
            def inner(t, m16):
                off = c * _CH + t * (16 * _U)
                for u in range(_U):
                    m16 = jnp.maximum(m16, rowbuf[pl.ds(off + u * 16, 16)])
                return m16
            m16 = lax.fori_loop(0, _CH // (16 * _U), inner, _splat_f(_NEG))
            cm = jnp.max(m16)
            cmax[c] = cm
            return jnp.maximum(G, cm)

        G = lax.fori_loop(0, _NCH, chunk_body, _NEG)

        G16 = _splat_f(G)

        def sum_body(t, s16):
            off = t * (16 * _U)
            for u in range(_U):
                s16 = s16 + jnp.exp(rowbuf[pl.ds(off + u * 16, 16)] - G16)
            return s16

        s16 = lax.fori_loop(0, _VOCAB // (16 * _U), sum_body, _splat_f(0.0))
        S = jnp.sum(s16)

        for _ in range(_K2):
            def scan_cmax(c, carry):
                bm, bc = carry
                v = cmax[c]
                better = v > bm
                return jnp.where(better, v, bm), jnp.where(better, c, bc)

            bm, bc = lax.fori_loop(0, _NCH, scan_cmax, (_NEG, jnp.int32(0)))
            base = bc * _CH
            bm16 = _splat_f(bm)

            def scan_pos(t, best):
                off = base + t * (16 * _U)
                for u in range(_U):
                    v = rowbuf[pl.ds(off + u * 16, 16)]
                    pos16 = iota16 + (off + u * 16)
                    best = jnp.minimum(
                        best, jnp.min(jnp.where(v == bm16, pos16, _BIGI)))
                return best

            pos = lax.fori_loop(0, _CH // (16 * _U), scan_pos, _BIGI)

            lane = pos % 16
            vecbase = pos - lane
            vv = rowbuf[pl.ds(vecbase, 16)]
            rowbuf[pl.ds(vecbase, 16)] = jnp.where(iota16 == lane, _NEG, vv)

            def rescan(t, m16):
                off = base + t * (16 * _U)
                for u in range(_U):
                    m16 = jnp.maximum(m16, rowbuf[pl.ds(off + u * 16, 16)])
                return m16

            m16 = lax.fori_loop(0, _CH // (16 * _U), rescan, _splat_f(_NEG))
            cmax[bc] = jnp.max(m16)

            vals_all.append(bm)
            flats_all.append(beam * _VOCAB + pos)

        g_list.append(G)
        s_list.append(S)

    v_lo = _splat_f(0.0)
    v_hi = _splat_f(0.0)
    f_lo = _splat_i(0)
    f_hi = _splat_i(0)
    for k in range(16):
        sel = iota16 == k
        v_lo = jnp.where(sel, _splat_f(vals_all[k]), v_lo)
        v_hi = jnp.where(sel, _splat_f(vals_all[16 + k]), v_hi)
        f_lo = jnp.where(sel, _splat_i(flats_all[k]), f_lo)
        f_hi = jnp.where(sel, _splat_i(flats_all[16 + k]), f_hi)
    gs_vec = _splat_f(0.0)
    for k in range(_BEAM):
        gs_vec = jnp.where(iota16 == k, _splat_f(g_list[k]), gs_vec)
        gs_vec = jnp.where(iota16 == (_BEAM + k), _splat_f(s_list[k]), gs_vec)

    stage_v[pl.ds(0, 16)] = v_lo
    stage_v[pl.ds(16, 16)] = v_hi
    stage_f[pl.ds(0, 16)] = f_lo
    stage_f[pl.ds(16, 16)] = f_hi
    stage_gs[pl.ds(0, 16)] = gs_vec

    pltpu.sync_copy(stage_v, out_v.at[b])
    pltpu.sync_copy(stage_f, out_f.at[b])
    pltpu.sync_copy(stage_gs, out_gs.at[b])


def _sc_topk(logits):
    mesh = plsc.VectorSubcoreMesh(core_axis_name="c", subcore_axis_name="s",
                                  num_cores=2, num_subcores=16)
    fn = pl.kernel(
        _sc_body,
        out_type=[
            jax.ShapeDtypeStruct((_BATCH, 2 * _K2 * 2), jnp.float32),
            jax.ShapeDtypeStruct((_BATCH, 2 * _K2 * 2), jnp.int32),
            jax.ShapeDtypeStruct((_BATCH, 16), jnp.float32),
        ],
        mesh=mesh,
        scratch_types=[
            pltpu.VMEM((_VOCAB,), jnp.float32),
            pltpu.SMEM((_NCH,), jnp.float32),
            pltpu.VMEM((2 * _K2 * 2,), jnp.float32),
            pltpu.VMEM((2 * _K2 * 2,), jnp.int32),
            pltpu.VMEM((16,), jnp.float32),
        ],
        compiler_params=pltpu.CompilerParams(needs_layout_passes=False),
    )
    return fn(logits)


def _tail_kernel(v_ref, f_ref, gs_ref, alp_ref, seq_ref, seq_out_ref, lp_out_ref):
    v = v_ref[...]
    f = f_ref[...]
    gs = gs_ref[...]
    alp = alp_ref[...]
    seq = seq_ref[...]

    n32 = 2 * _K2 * 2
    col = lax.broadcasted_iota(jnp.int32, (_BATCH, n32), 1)
    bcol = col // _K2
    Gx = jnp.zeros((_BATCH, n32), jnp.float32)
    Sx = jnp.zeros((_BATCH, n32), jnp.float32)
    Ax = jnp.zeros((_BATCH, n32), jnp.float32)
    for k in range(_BEAM):
        m = bcol == k
        Gx = jnp.where(m, gs[:, k:k + 1], Gx)
        Sx = jnp.where(m, gs[:, _BEAM + k:_BEAM + k + 1], Sx)
        Ax = jnp.where(m, alp[:, k:k + 1], Ax)
    score = v - (Gx + jnp.log(Sx)) + Ax

    NEGF = jnp.float32(-3.0e38)
    work = score
    cand_val = jnp.zeros((_BATCH, _K2), jnp.float32)
    cand_flat = jnp.zeros((_BATCH, _K2), jnp.int32)
    col8 = lax.broadcasted_iota(jnp.int32, (_BATCH, _K2), 1)
    for j in range(_K2):
        m = jnp.max(work, axis=1, keepdims=True)
        selflat = jnp.min(jnp.where(work == m, f, _BIGI), axis=1, keepdims=True)
        work = jnp.where(f == selflat, NEGF, work)
        cand_val = jnp.where(col8 == j, m, cand_val)
        cand_flat = jnp.where(col8 == j, selflat, cand_flat)

    topk_id = cand_flat % _VOCAB
    topk_beam = cand_flat // _VOCAB

    ts = jnp.zeros((_BATCH, _K2, seq.shape[2]), jnp.int32)
    bsel = topk_beam[:, :, None]
    for k in range(_BEAM):
        ts = jnp.where(jnp.broadcast_to(bsel == k, ts.shape),
                       jnp.broadcast_to(seq[:, k:k + 1, :], ts.shape), ts)
    topk_seq = jnp.concatenate([ts, topk_id[:, :, None]], axis=2)

    fin = (topk_id == _EOS_ID).astype(jnp.float32)
    masked = cand_val + fin * jnp.float32(-_NEG_INF)

    work2 = masked
    out_lp = jnp.zeros((_BATCH, _BEAM), jnp.float32)
    col4 = lax.broadcasted_iota(jnp.int32, (_BATCH, _BEAM), 1)
    out_seq = jnp.zeros((_BATCH, _BEAM, topk_seq.shape[2]), jnp.int32)
    row4 = lax.broadcasted_iota(jnp.int32, (_BATCH, _BEAM, 1), 1)
    for k in range(_BEAM):
        m2 = jnp.max(work2, axis=1, keepdims=True)
        selpos = jnp.min(jnp.where(work2 == m2, col8, _BIGI), axis=1,
                         keepdims=True)
        work2 = jnp.where(col8 == selpos, NEGF, work2)
        out_lp = jnp.where(col4 == k, m2, out_lp)
        rowk = jnp.zeros((_BATCH, 1, topk_seq.shape[2]), jnp.int32)
        for j in range(_K2):
            rowk = jnp.where(
                jnp.broadcast_to(selpos[:, :, None] == j, rowk.shape),
                topk_seq[:, j:j + 1, :], rowk)
        out_seq = jnp.where(jnp.broadcast_to(row4 == k, out_seq.shape),
                            jnp.broadcast_to(rowk, out_seq.shape), out_seq)

    seq_out_ref[...] = out_seq
    lp_out_ref[...] = out_lp


def kernel(logits, alive_log_probs, alive_seq):
    batch, beam, V = logits.shape
    cur_len = alive_seq.shape[2]

    v, f, gs = _sc_topk(logits)

    seq_out, lp_out = pl.pallas_call(
        _tail_kernel,
        out_shape=[
            jax.ShapeDtypeStruct((batch, beam, cur_len + 1), jnp.int32),
            jax.ShapeDtypeStruct((batch, beam), jnp.float32),
        ],
    )(v, f, gs, alive_log_probs, alive_seq)
    return seq_out, lp_out


# speedup vs baseline: 9.1898x; 1.0159x over previous
import functools

import jax
import jax.numpy as jnp
from jax import lax
from jax.experimental import pallas as pl
from jax.experimental.pallas import tpu as pltpu
from jax.experimental.pallas import tpu_sc as plsc

_EOS_ID = 2
_NEG_INF = 1.0e7

_BATCH = 32
_BEAM = 4
_VOCAB = 100000
_K2 = 8

_CH = 2000
_NCH = _VOCAB // _CH
_U = 5
_NEG = -3.0e38
_BIGI = 2**31 - 1


def _splat_f(x):
    return jnp.full((16,), x, jnp.float32)


def _splat_i(x):
    return jnp.full((16,), x, jnp.int32)


_NLANES = 5


def _worker_id():
    return lax.axis_index("s") * 2 + lax.axis_index("c")


def _put(buf, idx, x, iota16, dtype=jnp.float32):
    plsc.store_scatter(buf, [jnp.full((16,), idx, jnp.int32)],
                       jnp.full((16,), x, dtype), mask=iota16 == 0)


def _sc_body(logits, out_v, out_f, out_gs, rowbuf, cmaxv, stage_v, stage_f, stage_gs):
    b = _worker_id()
    iota16 = lax.iota(jnp.int32, 16)
    nvec = _CH // 16

    for beam in range(_BEAM):
        pltpu.sync_copy(logits.at[b, beam], rowbuf)

        for t in range(4):
            cmaxv[pl.ds(t * 16, 16)] = _splat_f(_NEG)

        def chunk_body(c, G):
            base = c * _CH
            accs = [_splat_f(_NEG) for _ in range(_NLANES)]
            for i in range(nvec):
                accs[i % _NLANES] = jnp.maximum(
                    accs[i % _NLANES], rowbuf[pl.ds(base + i * 16, 16)])
            m16 = accs[0]
            for a in accs[1:]:
                m16 = jnp.maximum(m16, a)
            cm = jnp.max(m16)
            _put(cmaxv, c, cm, iota16)
            return jnp.maximum(G, cm)

        G = lax.fori_loop(0, _NCH, chunk_body, jnp.float32(_NEG))

        G16 = _splat_f(G)

        def sum_body(c, s):
            base = c * _CH
            accs = [_splat_f(0.0) for _ in range(_NLANES)]
            for i in range(nvec):
                accs[i % _NLANES] = accs[i % _NLANES] + jnp.exp(
                    rowbuf[pl.ds(base + i * 16, 16)] - G16)
            s16 = accs[0]
            for a in accs[1:]:
                s16 = s16 + a
            return s + jnp.sum(s16)

        S = lax.fori_loop(0, _NCH, sum_body, jnp.float32(0.0))

        for k in range(_K2):
            m16 = _splat_f(_NEG)
            am16 = _splat_i(0)
            for t in range(4):
                v = cmaxv[pl.ds(t * 16, 16)]
                idx16 = iota16 + t * 16
                better = v > m16
                m16 = jnp.where(better, v, m16)
                am16 = jnp.where(better, idx16, am16)
            bm = jnp.max(m16)
            bc = jnp.min(jnp.where(m16 == bm, am16, _BIGI))
            base = bc * _CH
            bm16 = _splat_f(bm)

            def scan_pos(t, best):
                off = base + t * (16 * _U)
                for u in range(_U):
                    v = rowbuf[pl.ds(off + u * 16, 16)]
                    pos16 = iota16 + (off + u * 16)
                    best = jnp.minimum(
                        best, jnp.min(jnp.where(v == bm16, pos16, _BIGI)))
                return best

            pos = lax.fori_loop(0, nvec // _U, scan_pos, _BIGI)

            lane = pos % 16
            vecbase = pos - lane
            vv = rowbuf[pl.ds(vecbase, 16)]
            rowbuf[pl.ds(vecbase, 16)] = jnp.where(iota16 == lane, _NEG, vv)

            def rescan(t, m):
                off = base + t * (16 * _U)
                for u in range(_U):
                    m = jnp.maximum(m, rowbuf[pl.ds(off + u * 16, 16)])
                return m

            nm16 = lax.fori_loop(0, nvec // _U, rescan, _splat_f(_NEG))
            _put(cmaxv, bc, jnp.max(nm16), iota16)

            _put(stage_v, beam * _K2 + k, bm, iota16)
            _put(stage_f, beam * _K2 + k, beam * _VOCAB + pos, iota16,
                 jnp.int32)

        _put(stage_gs, beam, G, iota16)
        _put(stage_gs, _BEAM + beam, S, iota16)

    pltpu.sync_copy(stage_v, out_v.at[b])
    pltpu.sync_copy(stage_f, out_f.at[b])
    pltpu.sync_copy(stage_gs, out_gs.at[b])


def _sc_topk(logits):
    mesh = plsc.VectorSubcoreMesh(core_axis_name="c", subcore_axis_name="s",
                                  num_cores=2, num_subcores=16)
    fn = pl.kernel(
        _sc_body,
        out_type=[
            jax.ShapeDtypeStruct((_BATCH, 2 * _K2 * 2), jnp.float32),
            jax.ShapeDtypeStruct((_BATCH, 2 * _K2 * 2), jnp.int32),
            jax.ShapeDtypeStruct((_BATCH, 16), jnp.float32),
        ],
        mesh=mesh,
        scratch_types=[
            pltpu.VMEM((_VOCAB,), jnp.float32),
            pltpu.VMEM((64,), jnp.float32),
            pltpu.VMEM((2 * _K2 * 2,), jnp.float32),
            pltpu.VMEM((2 * _K2 * 2,), jnp.int32),
            pltpu.VMEM((16,), jnp.float32),
        ],
        compiler_params=pltpu.CompilerParams(needs_layout_passes=False),
    )
    return fn(logits)


def _tail_kernel(v_ref, f_ref, gs_ref, alp_ref, seq_ref, seq_out_ref, lp_out_ref):
    v = v_ref[...]
    f = f_ref[...]
    gs = gs_ref[...]
    alp = alp_ref[...]
    seq = seq_ref[...]

    n32 = 2 * _K2 * 2
    col = lax.broadcasted_iota(jnp.int32, (_BATCH, n32), 1)
    bcol = col // _K2
    Gx = jnp.zeros((_BATCH, n32), jnp.float32)
    Sx = jnp.zeros((_BATCH, n32), jnp.float32)
    Ax = jnp.zeros((_BATCH, n32), jnp.float32)
    for k in range(_BEAM):
        m = bcol == k
        Gx = jnp.where(m, gs[:, k:k + 1], Gx)
        Sx = jnp.where(m, gs[:, _BEAM + k:_BEAM + k + 1], Sx)
        Ax = jnp.where(m, alp[:, k:k + 1], Ax)
    score = v - (Gx + jnp.log(Sx)) + Ax

    NEGF = jnp.float32(-3.0e38)
    work = score
    cand_val = jnp.zeros((_BATCH, _K2), jnp.float32)
    cand_flat = jnp.zeros((_BATCH, _K2), jnp.int32)
    col8 = lax.broadcasted_iota(jnp.int32, (_BATCH, _K2), 1)
    for j in range(_K2):
        m = jnp.max(work, axis=1, keepdims=True)
        selflat = jnp.min(jnp.where(work == m, f, _BIGI), axis=1, keepdims=True)
        work = jnp.where(f == selflat, NEGF, work)
        cand_val = jnp.where(col8 == j, m, cand_val)
        cand_flat = jnp.where(col8 == j, selflat, cand_flat)

    topk_id = cand_flat % _VOCAB
    topk_beam = cand_flat // _VOCAB

    ts = jnp.zeros((_BATCH, _K2, seq.shape[2]), jnp.int32)
    bsel = topk_beam[:, :, None]
    for k in range(_BEAM):
        ts = jnp.where(jnp.broadcast_to(bsel == k, ts.shape),
                       jnp.broadcast_to(seq[:, k:k + 1, :], ts.shape), ts)
    topk_seq = jnp.concatenate([ts, topk_id[:, :, None]], axis=2)

    fin = (topk_id == _EOS_ID).astype(jnp.float32)
    masked = cand_val + fin * jnp.float32(-_NEG_INF)

    work2 = masked
    out_lp = jnp.zeros((_BATCH, _BEAM), jnp.float32)
    col4 = lax.broadcasted_iota(jnp.int32, (_BATCH, _BEAM), 1)
    out_seq = jnp.zeros((_BATCH, _BEAM, topk_seq.shape[2]), jnp.int32)
    row4 = lax.broadcasted_iota(jnp.int32, (_BATCH, _BEAM, 1), 1)
    for k in range(_BEAM):
        m2 = jnp.max(work2, axis=1, keepdims=True)
        selpos = jnp.min(jnp.where(work2 == m2, col8, _BIGI), axis=1,
                         keepdims=True)
        work2 = jnp.where(col8 == selpos, NEGF, work2)
        out_lp = jnp.where(col4 == k, m2, out_lp)
        rowk = jnp.zeros((_BATCH, 1, topk_seq.shape[2]), jnp.int32)
        for j in range(_K2):
            rowk = jnp.where(
                jnp.broadcast_to(selpos[:, :, None] == j, rowk.shape),
                topk_seq[:, j:j + 1, :], rowk)
        out_seq = jnp.where(jnp.broadcast_to(row4 == k, out_seq.shape),
                            jnp.broadcast_to(rowk, out_seq.shape), out_seq)

    seq_out_ref[...] = out_seq
    lp_out_ref[...] = out_lp


def kernel(logits, alive_log_probs, alive_seq):
    batch, beam, V = logits.shape
    cur_len = alive_seq.shape[2]

    v, f, gs = _sc_topk(logits)

    seq_out, lp_out = pl.pallas_call(
        _tail_kernel,
        out_shape=[
            jax.ShapeDtypeStruct((batch, beam, cur_len + 1), jnp.int32),
            jax.ShapeDtypeStruct((batch, beam), jnp.float32),
        ],
    )(v, f, gs, alive_log_probs, alive_seq)
    return seq_out, lp_out


# speedup vs baseline: 9.3931x; 1.0221x over previous
import functools

import jax
import jax.numpy as jnp
from jax import lax
from jax.experimental import pallas as pl
from jax.experimental.pallas import tpu as pltpu
from jax.experimental.pallas import tpu_sc as plsc

_EOS_ID = 2
_NEG_INF = 1.0e7

_BATCH = 32
_BEAM = 4
_VOCAB = 100000
_K2 = 8

_CH = 2000
_NCH = _VOCAB // _CH
_U = 5
_NEG = -3.0e38
_BIGI = 2**31 - 1


def _splat_f(x):
    return jnp.full((16,), x, jnp.float32)


def _splat_i(x):
    return jnp.full((16,), x, jnp.int32)


_NLANES = 5


def _worker_id():
    return lax.axis_index("s") * 2 + lax.axis_index("c")


def _put(buf, idx, x, iota16, dtype=jnp.float32):
    plsc.store_scatter(buf, [jnp.full((16,), idx, jnp.int32)],
                       jnp.full((16,), x, dtype), mask=iota16 == 0)


def _sc_body(logits, out_v, out_f, out_gs, rowbuf, cmaxv, cmaxs, stage_v,
             stage_f, stage_gs):
    b = _worker_id()
    iota16 = lax.iota(jnp.int32, 16)
    nvec = _CH // 16

    for beam in range(_BEAM):
        pltpu.sync_copy(logits.at[b, beam], rowbuf)

        def chunk_body(c, g16):
            base = c * _CH
            accs = [_splat_f(_NEG) for _ in range(_NLANES)]
            for i in range(nvec):
                accs[i % _NLANES] = jnp.maximum(
                    accs[i % _NLANES], rowbuf[pl.ds(base + i * 16, 16)])
            m16 = accs[0]
            for a in accs[1:]:
                m16 = jnp.maximum(m16, a)
            cmaxv[pl.ds(c * 16, 16)] = m16
            return jnp.maximum(g16, m16)

        g16 = lax.fori_loop(0, _NCH, chunk_body, _splat_f(_NEG))
        G = jnp.max(g16)

        for g in range(4):
            acc = _splat_f(_NEG)
            for l in range(16):
                gi = (iota16 + g * 16) * 16 + l
                acc = jnp.maximum(acc, plsc.load_gather(cmaxv, [gi]))
            if g == 3:
                acc = jnp.where(iota16 + g * 16 < _NCH, acc, _NEG)
            cmaxs[pl.ds(g * 16, 16)] = acc

        G16 = _splat_f(G)

        def sum_body(c, accs):
            base = c * _CH
            accs = list(accs)
            for i in range(nvec):
                accs[i % _NLANES] = accs[i % _NLANES] + jnp.exp(
                    rowbuf[pl.ds(base + i * 16, 16)] - G16)
            return tuple(accs)

        accs = lax.fori_loop(0, _NCH, sum_body,
                             tuple(_splat_f(0.0) for _ in range(_NLANES)))
        s16 = accs[0]
        for a in accs[1:]:
            s16 = s16 + a
        S = jnp.sum(s16)

        for k in range(_K2):
            m16 = _splat_f(_NEG)
            am16 = _splat_i(0)
            for t in range(4):
                v = cmaxs[pl.ds(t * 16, 16)]
                idx16 = iota16 + t * 16
                better = v > m16
                m16 = jnp.where(better, v, m16)
                am16 = jnp.where(better, idx16, am16)
            bm = jnp.max(m16)
            bc = jnp.min(jnp.where(m16 == bm, am16, _BIGI))
            base = bc * _CH
            bm16 = _splat_f(bm)

            def scan_pos(t, best16):
                off = base + t * (16 * _U)
                for u in range(_U):
                    v = rowbuf[pl.ds(off + u * 16, 16)]
                    pos16 = iota16 + (off + u * 16)
                    best16 = jnp.minimum(
                        best16, jnp.where(v == bm16, pos16, _BIGI))
                return best16

            best16 = lax.fori_loop(0, nvec // _U, scan_pos, _splat_i(_BIGI))
            pos = jnp.min(best16)

            lane = pos % 16
            vecbase = pos - lane
            vv = rowbuf[pl.ds(vecbase, 16)]
            rowbuf[pl.ds(vecbase, 16)] = jnp.where(iota16 == lane, _NEG, vv)

            def rescan(t, m):
                off = base + t * (16 * _U)
                for u in range(_U):
                    m = jnp.maximum(m, rowbuf[pl.ds(off + u * 16, 16)])
                return m

            nm16 = lax.fori_loop(0, nvec // _U, rescan, _splat_f(_NEG))
            _put(cmaxs, bc, jnp.max(nm16), iota16)

            _put(stage_v, beam * _K2 + k, bm, iota16)
            _put(stage_f, beam * _K2 + k, beam * _VOCAB + pos, iota16,
                 jnp.int32)

        _put(stage_gs, beam, G, iota16)
        _put(stage_gs, _BEAM + beam, S, iota16)

    pltpu.sync_copy(stage_v, out_v.at[b])
    pltpu.sync_copy(stage_f, out_f.at[b])
    pltpu.sync_copy(stage_gs, out_gs.at[b])


def _sc_topk(logits):
    mesh = plsc.VectorSubcoreMesh(core_axis_name="c", subcore_axis_name="s",
                                  num_cores=2, num_subcores=16)
    fn = pl.kernel(
        _sc_body,
        out_type=[
            jax.ShapeDtypeStruct((_BATCH, 2 * _K2 * 2), jnp.float32),
            jax.ShapeDtypeStruct((_BATCH, 2 * _K2 * 2), jnp.int32),
            jax.ShapeDtypeStruct((_BATCH, 16), jnp.float32),
        ],
        mesh=mesh,
        scratch_types=[
            pltpu.VMEM((_VOCAB,), jnp.float32),
            pltpu.VMEM((1024,), jnp.float32),
            pltpu.VMEM((64,), jnp.float32),
            pltpu.VMEM((2 * _K2 * 2,), jnp.float32),
            pltpu.VMEM((2 * _K2 * 2,), jnp.int32),
            pltpu.VMEM((16,), jnp.float32),
        ],
        compiler_params=pltpu.CompilerParams(needs_layout_passes=False),
    )
    return fn(logits)


def _tail_kernel(v_ref, f_ref, gs_ref, alp_ref, seq_ref, seq_out_ref, lp_out_ref):
    v = v_ref[...]
    f = f_ref[...]
    gs = gs_ref[...]
    alp = alp_ref[...]
    seq = seq_ref[...]

    n32 = 2 * _K2 * 2
    col = lax.broadcasted_iota(jnp.int32, (_BATCH, n32), 1)
    bcol = col // _K2
    Gx = jnp.zeros((_BATCH, n32), jnp.float32)
    Sx = jnp.zeros((_BATCH, n32), jnp.float32)
    Ax = jnp.zeros((_BATCH, n32), jnp.float32)
    for k in range(_BEAM):
        m = bcol == k
        Gx = jnp.where(m, gs[:, k:k + 1], Gx)
        Sx = jnp.where(m, gs[:, _BEAM + k:_BEAM + k + 1], Sx)
        Ax = jnp.where(m, alp[:, k:k + 1], Ax)
    score = v - (Gx + jnp.log(Sx)) + Ax

    NEGF = jnp.float32(-3.0e38)
    work = score
    cand_val = jnp.zeros((_BATCH, _K2), jnp.float32)
    cand_flat = jnp.zeros((_BATCH, _K2), jnp.int32)
    col8 = lax.broadcasted_iota(jnp.int32, (_BATCH, _K2), 1)
    for j in range(_K2):
        m = jnp.max(work, axis=1, keepdims=True)
        selflat = jnp.min(jnp.where(work == m, f, _BIGI), axis=1, keepdims=True)
        work = jnp.where(f == selflat, NEGF, work)
        cand_val = jnp.where(col8 == j, m, cand_val)
        cand_flat = jnp.where(col8 == j, selflat, cand_flat)

    topk_id = cand_flat % _VOCAB
    topk_beam = cand_flat // _VOCAB

    ts = jnp.zeros((_BATCH, _K2, seq.shape[2]), jnp.int32)
    bsel = topk_beam[:, :, None]
    for k in range(_BEAM):
        ts = jnp.where(jnp.broadcast_to(bsel == k, ts.shape),
                       jnp.broadcast_to(seq[:, k:k + 1, :], ts.shape), ts)
    topk_seq = jnp.concatenate([ts, topk_id[:, :, None]], axis=2)

    fin = (topk_id == _EOS_ID).astype(jnp.float32)
    masked = cand_val + fin * jnp.float32(-_NEG_INF)

    work2 = masked
    out_lp = jnp.zeros((_BATCH, _BEAM), jnp.float32)
    col4 = lax.broadcasted_iota(jnp.int32, (_BATCH, _BEAM), 1)
    out_seq = jnp.zeros((_BATCH, _BEAM, topk_seq.shape[2]), jnp.int32)
    row4 = lax.broadcasted_iota(jnp.int32, (_BATCH, _BEAM, 1), 1)
    for k in range(_BEAM):
        m2 = jnp.max(work2, axis=1, keepdims=True)
        selpos = jnp.min(jnp.where(work2 == m2, col8, _BIGI), axis=1,
                         keepdims=True)
        work2 = jnp.where(col8 == selpos, NEGF, work2)
        out_lp = jnp.where(col4 == k, m2, out_lp)
        rowk = jnp.zeros((_BATCH, 1, topk_seq.shape[2]), jnp.int32)
        for j in range(_K2):
            rowk = jnp.where(
                jnp.broadcast_to(selpos[:, :, None] == j, rowk.shape),
                topk_seq[:, j:j + 1, :], rowk)
        out_seq = jnp.where(jnp.broadcast_to(row4 == k, out_seq.shape),
                            jnp.broadcast_to(rowk, out_seq.shape), out_seq)

    seq_out_ref[...] = out_seq
    lp_out_ref[...] = out_lp


def kernel(logits, alive_log_probs, alive_seq):
    batch, beam, V = logits.shape
    cur_len = alive_seq.shape[2]

    v, f, gs = _sc_topk(logits)

    seq_out, lp_out = pl.pallas_call(
        _tail_kernel,
        out_shape=[
            jax.ShapeDtypeStruct((batch, beam, cur_len + 1), jnp.int32),
            jax.ShapeDtypeStruct((batch, beam), jnp.float32),
        ],
    )(v, f, gs, alive_log_probs, alive_seq)
    return seq_out, lp_out


# speedup vs baseline: 11.8034x; 1.2566x over previous
import functools

import jax
import jax.numpy as jnp
from jax import lax
from jax.experimental import pallas as pl
from jax.experimental.pallas import tpu as pltpu
from jax.experimental.pallas import tpu_sc as plsc

_EOS_ID = 2
_NEG_INF = 1.0e7

_BATCH = 32
_BEAM = 4
_VOCAB = 100000
_K2 = 8

_CH = 2000
_NCH = _VOCAB // _CH
_U = 5
_NEG = -3.0e38
_BIGI = 2**31 - 1


def _splat_f(x):
    return jnp.full((16,), x, jnp.float32)


def _splat_i(x):
    return jnp.full((16,), x, jnp.int32)


_NLANES = 5


def _worker_id():
    return lax.axis_index("s") * 2 + lax.axis_index("c")


def _put(buf, idx, x, iota16, dtype=jnp.float32):
    plsc.store_scatter(buf, [jnp.full((16,), idx, jnp.int32)],
                       jnp.full((16,), x, dtype), mask=iota16 == 0)


def _sc_body(logits, out_v, out_f, rowbuf, cmaxv, cmaxs, stage_v, stage_f):
    b = _worker_id()
    iota16 = lax.iota(jnp.int32, 16)
    nvec = _CH // 16

    for beam in range(_BEAM):
        pltpu.sync_copy(logits.at[b, beam], rowbuf)

        def chunk_body(c, g16):
            base = c * _CH
            accs = [_splat_f(_NEG) for _ in range(_NLANES)]
            for i in range(nvec):
                accs[i % _NLANES] = jnp.maximum(
                    accs[i % _NLANES], rowbuf[pl.ds(base + i * 16, 16)])
            m16 = accs[0]
            for a in accs[1:]:
                m16 = jnp.maximum(m16, a)
            cmaxv[pl.ds(c * 16, 16)] = m16
            return jnp.maximum(g16, m16)

        lax.fori_loop(0, _NCH, chunk_body, _splat_f(_NEG))

        for g in range(4):
            acc = _splat_f(_NEG)
            for l in range(16):
                gi = (iota16 + g * 16) * 16 + l
                acc = jnp.maximum(acc, plsc.load_gather(cmaxv, [gi]))
            if g == 3:
                acc = jnp.where(iota16 + g * 16 < _NCH, acc, _NEG)
            cmaxs[pl.ds(g * 16, 16)] = acc

        for k in range(_K2):
            m16 = _splat_f(_NEG)
            am16 = _splat_i(0)
            for t in range(4):
                v = cmaxs[pl.ds(t * 16, 16)]
                idx16 = iota16 + t * 16
                better = v > m16
                m16 = jnp.where(better, v, m16)
                am16 = jnp.where(better, idx16, am16)
            bm = jnp.max(m16)
            bc = jnp.min(jnp.where(m16 == bm, am16, _BIGI))
            base = bc * _CH
            bm16 = _splat_f(bm)

            def scan_pos(t, best16):
                off = base + t * (16 * _U)
                for u in range(_U):
                    v = rowbuf[pl.ds(off + u * 16, 16)]
                    pos16 = iota16 + (off + u * 16)
                    best16 = jnp.minimum(
                        best16, jnp.where(v == bm16, pos16, _BIGI))
                return best16

            best16 = lax.fori_loop(0, nvec // _U, scan_pos, _splat_i(_BIGI))
            pos = jnp.min(best16)

            lane = pos % 16
            vecbase = pos - lane
            vv = rowbuf[pl.ds(vecbase, 16)]
            rowbuf[pl.ds(vecbase, 16)] = jnp.where(iota16 == lane, _NEG, vv)

            def rescan(t, m):
                off = base + t * (16 * _U)
                for u in range(_U):
                    m = jnp.maximum(m, rowbuf[pl.ds(off + u * 16, 16)])
                return m

            nm16 = lax.fori_loop(0, nvec // _U, rescan, _splat_f(_NEG))
            _put(cmaxs, bc, jnp.max(nm16), iota16)

            _put(stage_v, beam * _K2 + k, bm, iota16)
            _put(stage_f, beam * _K2 + k, beam * _VOCAB + pos, iota16,
                 jnp.int32)

    pltpu.sync_copy(stage_v, out_v.at[b])
    pltpu.sync_copy(stage_f, out_f.at[b])


def _sc_topk(logits):
    mesh = plsc.VectorSubcoreMesh(core_axis_name="c", subcore_axis_name="s",
                                  num_cores=2, num_subcores=16)
    fn = pl.kernel(
        _sc_body,
        out_type=[
            jax.ShapeDtypeStruct((_BATCH, 2 * _K2 * 2), jnp.float32),
            jax.ShapeDtypeStruct((_BATCH, 2 * _K2 * 2), jnp.int32),
        ],
        mesh=mesh,
        scratch_types=[
            pltpu.VMEM((_VOCAB,), jnp.float32),
            pltpu.VMEM((1024,), jnp.float32),
            pltpu.VMEM((64,), jnp.float32),
            pltpu.VMEM((2 * _K2 * 2,), jnp.float32),
            pltpu.VMEM((2 * _K2 * 2,), jnp.int32),
        ],
        compiler_params=pltpu.CompilerParams(needs_layout_passes=False),
    )
    return fn(logits)


def _lse_kernel(x_ref, lse_ref):
    x = x_ref[0]
    m = jnp.max(x, axis=1, keepdims=True)
    s = jnp.sum(jnp.exp(x - m), axis=1, keepdims=True)
    lse_ref[0] = (m + jnp.log(s)).reshape(1, x.shape[0])


def _lse(logits):
    batch, beam, V = logits.shape
    out = pl.pallas_call(
        _lse_kernel,
        grid=(batch,),
        in_specs=[pl.BlockSpec((1, beam, V), lambda b: (b, 0, 0))],
        out_specs=pl.BlockSpec((1, 1, beam), lambda b: (b, 0, 0)),
        out_shape=jax.ShapeDtypeStruct((batch, 1, beam), jnp.float32),
    )(logits)
    return out.reshape(batch, beam)


def _tail_kernel(v_ref, f_ref, lse_ref, alp_ref, seq_ref, seq_out_ref, lp_out_ref):
    v = v_ref[...]
    f = f_ref[...]
    lse = lse_ref[...]
    alp = alp_ref[...]
    seq = seq_ref[...]

    n32 = 2 * _K2 * 2
    col = lax.broadcasted_iota(jnp.int32, (_BATCH, n32), 1)
    bcol = col // _K2
    Lx = jnp.zeros((_BATCH, n32), jnp.float32)
    Ax = jnp.zeros((_BATCH, n32), jnp.float32)
    for k in range(_BEAM):
        m = bcol == k
        Lx = jnp.where(m, lse[:, k:k + 1], Lx)
        Ax = jnp.where(m, alp[:, k:k + 1], Ax)
    score = v - Lx + Ax

    NEGF = jnp.float32(-3.0e38)
    work = score
    cand_val = jnp.zeros((_BATCH, _K2), jnp.float32)
    cand_flat = jnp.zeros((_BATCH, _K2), jnp.int32)
    col8 = lax.broadcasted_iota(jnp.int32, (_BATCH, _K2), 1)
    for j in range(_K2):
        m = jnp.max(work, axis=1, keepdims=True)
        selflat = jnp.min(jnp.where(work == m, f, _BIGI), axis=1, keepdims=True)
        work = jnp.where(f == selflat, NEGF, work)
        cand_val = jnp.where(col8 == j, m, cand_val)
        cand_flat = jnp.where(col8 == j, selflat, cand_flat)

    topk_id = cand_flat % _VOCAB
    topk_beam = cand_flat // _VOCAB

    ts = jnp.zeros((_BATCH, _K2, seq.shape[2]), jnp.int32)
    bsel = topk_beam[:, :, None]
    for k in range(_BEAM):
        ts = jnp.where(jnp.broadcast_to(bsel == k, ts.shape),
                       jnp.broadcast_to(seq[:, k:k + 1, :], ts.shape), ts)
    topk_seq = jnp.concatenate([ts, topk_id[:, :, None]], axis=2)

    fin = (topk_id == _EOS_ID).astype(jnp.float32)
    masked = cand_val + fin * jnp.float32(-_NEG_INF)

    work2 = masked
    out_lp = jnp.zeros((_BATCH, _BEAM), jnp.float32)
    col4 = lax.broadcasted_iota(jnp.int32, (_BATCH, _BEAM), 1)
    out_seq = jnp.zeros((_BATCH, _BEAM, topk_seq.shape[2]), jnp.int32)
    row4 = lax.broadcasted_iota(jnp.int32, (_BATCH, _BEAM, 1), 1)
    for k in range(_BEAM):
        m2 = jnp.max(work2, axis=1, keepdims=True)
        selpos = jnp.min(jnp.where(work2 == m2, col8, _BIGI), axis=1,
                         keepdims=True)
        work2 = jnp.where(col8 == selpos, NEGF, work2)
        out_lp = jnp.where(col4 == k, m2, out_lp)
        rowk = jnp.zeros((_BATCH, 1, topk_seq.shape[2]), jnp.int32)
        for j in range(_K2):
            rowk = jnp.where(
                jnp.broadcast_to(selpos[:, :, None] == j, rowk.shape),
                topk_seq[:, j:j + 1, :], rowk)
        out_seq = jnp.where(jnp.broadcast_to(row4 == k, out_seq.shape),
                            jnp.broadcast_to(rowk, out_seq.shape), out_seq)

    seq_out_ref[...] = out_seq
    lp_out_ref[...] = out_lp


def kernel(logits, alive_log_probs, alive_seq):
    batch, beam, V = logits.shape
    cur_len = alive_seq.shape[2]

    v, f = _sc_topk(logits)
    lse = _lse(logits)

    seq_out, lp_out = pl.pallas_call(
        _tail_kernel,
        out_shape=[
            jax.ShapeDtypeStruct((batch, beam, cur_len + 1), jnp.int32),
            jax.ShapeDtypeStruct((batch, beam), jnp.float32),
        ],
    )(v, f, lse, alive_log_probs, alive_seq)
    return seq_out, lp_out
